# Initial kernel scaffold; baseline (speedup 1.0000x reference)
import jax
import jax.numpy as jnp
from jax.experimental import pallas as pl


def kernel(z, edge_index, edge_type, rel_emb):
    raise NotImplementedError("write your pallas kernel here")


# speedup vs baseline: 2.5120x; 2.5120x over previous
import jax
import jax.numpy as jnp
from jax import lax
from jax.experimental import pallas as pl
from jax.experimental.pallas import tpu as pltpu
from jax.experimental.pallas import tpu_sc as plsc

N_NODES = 10000
N_EDGES = 160000
HIDDEN = 256
NUM_REL = 1024
NSLICE = HIDDEN // 16

NW = 32
E_PER_W = N_EDGES // NW
CHUNK = 64
N_FULL = E_PER_W // CHUNK
TAIL = E_PER_W - N_FULL * CHUNK
OUT_PAD = N_FULL * CHUNK + 16


def _sc_kernel(src_hbm, dst_hbm, typ_hbm, z_hbm, rel_hbm, out_hbm,
               idx_s, idx_d, idx_t, idx_s8, idx_d8, idx_t8,
               rows_s, rows_d, rows_r, part, out_v, sem):
    wid = lax.axis_index("s") * 2 + lax.axis_index("c")
    base = wid * E_PER_W
    lanes16 = lax.iota(jnp.int32, 16) * 16

    def edge_body(e, _):
        sl0 = pl.ds(0, 16)
        acc = rows_s[e, sl0] * rows_r[e, sl0] * rows_d[e, sl0]
        for k in range(1, NSLICE):
            sl = pl.ds(k * 16, 16)
            acc = acc + rows_s[e, sl] * rows_r[e, sl] * rows_d[e, sl]
        part[pl.ds(e * 16, 16)] = acc
        return 0

    def reduce_groups(out_off, n_groups):
        for g in range(n_groups):
            acc = plsc.load_gather(part, [lanes16 + g * 256])
            for k in range(1, 16):
                acc = acc + plsc.load_gather(part, [lanes16 + (g * 256 + k)])
            out_v[pl.ds(out_off + g * 16, 16)] = acc

    def chunk_body(c, _):
        off = base + c * CHUNK
        pltpu.sync_copy(src_hbm.at[pl.ds(off, CHUNK)], idx_s)
        pltpu.sync_copy(dst_hbm.at[pl.ds(off, CHUNK)], idx_d)
        pltpu.sync_copy(typ_hbm.at[pl.ds(off, CHUNK)], idx_t)
        cp1 = pltpu.async_copy(z_hbm.at[idx_s], rows_s, sem)
        cp2 = pltpu.async_copy(z_hbm.at[idx_d], rows_d, sem)
        cp3 = pltpu.async_copy(rel_hbm.at[idx_t], rows_r, sem)
        cp1.wait()
        cp2.wait()
        cp3.wait()
        lax.fori_loop(0, CHUNK, edge_body, 0)
        reduce_groups(c * CHUNK, CHUNK // 16)
        return 0

    lax.fori_loop(0, N_FULL, chunk_body, 0)

    toff = base + N_FULL * CHUNK
    pltpu.sync_copy(src_hbm.at[pl.ds(toff, TAIL)], idx_s8)
    pltpu.sync_copy(dst_hbm.at[pl.ds(toff, TAIL)], idx_d8)
    pltpu.sync_copy(typ_hbm.at[pl.ds(toff, TAIL)], idx_t8)
    cp1 = pltpu.async_copy(z_hbm.at[idx_s8], rows_s.at[pl.ds(0, TAIL)], sem)
    cp2 = pltpu.async_copy(z_hbm.at[idx_d8], rows_d.at[pl.ds(0, TAIL)], sem)
    cp3 = pltpu.async_copy(rel_hbm.at[idx_t8], rows_r.at[pl.ds(0, TAIL)], sem)
    cp1.wait()
    cp2.wait()
    cp3.wait()
    lax.fori_loop(0, TAIL, edge_body, 0)
    reduce_groups(N_FULL * CHUNK, 1)

    pltpu.sync_copy(out_v.at[pl.ds(0, E_PER_W)],
                    out_hbm.at[pl.ds(base, E_PER_W)])


@jax.jit
def _dist_mult(src, dst, typ, z, rel_emb):
    mesh = plsc.VectorSubcoreMesh(core_axis_name="c", subcore_axis_name="s")
    f = pl.kernel(
        _sc_kernel,
        out_type=jax.ShapeDtypeStruct((N_EDGES,), jnp.float32),
        mesh=mesh,
        scratch_types=[
            pltpu.VMEM((CHUNK,), jnp.int32),
            pltpu.VMEM((CHUNK,), jnp.int32),
            pltpu.VMEM((CHUNK,), jnp.int32),
            pltpu.VMEM((TAIL,), jnp.int32),
            pltpu.VMEM((TAIL,), jnp.int32),
            pltpu.VMEM((TAIL,), jnp.int32),
            pltpu.VMEM((CHUNK, HIDDEN), jnp.float32),
            pltpu.VMEM((CHUNK, HIDDEN), jnp.float32),
            pltpu.VMEM((CHUNK, HIDDEN), jnp.float32),
            pltpu.VMEM((CHUNK * 16,), jnp.float32),
            pltpu.VMEM((OUT_PAD,), jnp.float32),
            pltpu.SemaphoreType.DMA,
        ],
        compiler_params=pltpu.CompilerParams(needs_layout_passes=False),
    )
    return f(src, dst, typ, z, rel_emb)


def kernel(z, edge_index, edge_type, rel_emb):
    edge_index = edge_index.astype(jnp.int32)
    edge_type = edge_type.astype(jnp.int32)
    return _dist_mult(edge_index[0], edge_index[1], edge_type, z, rel_emb)


# speedup vs baseline: 5.4629x; 2.1747x over previous
import jax
import jax.numpy as jnp
from jax import lax
from jax.experimental import pallas as pl
from jax.experimental.pallas import tpu as pltpu
from jax.experimental.pallas import tpu_sc as plsc

N_NODES = 10000
N_EDGES = 160000
HIDDEN = 256
NUM_REL = 1024
NSLICE = HIDDEN // 16

NW = 32
E_PER_W = N_EDGES // NW
CHUNK = 64
N_FULL = E_PER_W // CHUNK
TAIL = E_PER_W - N_FULL * CHUNK
OUT_PAD = N_FULL * CHUNK + 16


def _sc_kernel(src_hbm, dst_hbm, typ_hbm, z_hbm, rel_hbm, out_hbm,
               idx_all_s, idx_all_d, idx_all_t,
               rs0, rd0, rr0, rs1, rd1, rr1,
               part, out_v, sem0, sem1):
    wid = lax.axis_index("s") * 2 + lax.axis_index("c")
    base = wid * E_PER_W
    lanes16 = lax.iota(jnp.int32, 16) * 16

    def issue(c, rs, rd, rr, sem):
        o = c * CHUNK
        pltpu.async_copy(z_hbm.at[idx_all_s.at[pl.ds(o, CHUNK)]], rs, sem)
        pltpu.async_copy(z_hbm.at[idx_all_d.at[pl.ds(o, CHUNK)]], rd, sem)
        pltpu.async_copy(rel_hbm.at[idx_all_t.at[pl.ds(o, CHUNK)]], rr, sem)

    def drain(rs, rd, rr, sem):
        pltpu.make_async_copy(z_hbm.at[idx_all_s.at[pl.ds(0, CHUNK)]], rs, sem).wait()
        pltpu.make_async_copy(z_hbm.at[idx_all_d.at[pl.ds(0, CHUNK)]], rd, sem).wait()
        pltpu.make_async_copy(rel_hbm.at[idx_all_t.at[pl.ds(0, CHUNK)]], rr, sem).wait()

    def compute(rs, rd, rr, out_off, n_edges, n_groups):
        def edge_body(e, _):
            sl0 = pl.ds(0, 16)
            acc = rs[e, sl0] * rr[e, sl0] * rd[e, sl0]
            for k in range(1, NSLICE):
                sl = pl.ds(k * 16, 16)
                acc = acc + rs[e, sl] * rr[e, sl] * rd[e, sl]
            part[pl.ds(e * 16, 16)] = acc
            return 0

        lax.fori_loop(0, n_edges, edge_body, 0)
        for g in range(n_groups):
            acc = plsc.load_gather(part, [lanes16 + g * 256])
            for k in range(1, 16):
                acc = acc + plsc.load_gather(part, [lanes16 + (g * 256 + k)])
            out_v[pl.ds(out_off + g * 16, 16)] = acc

    pltpu.sync_copy(src_hbm.at[pl.ds(base, E_PER_W)], idx_all_s)
    pltpu.sync_copy(dst_hbm.at[pl.ds(base, E_PER_W)], idx_all_d)
    pltpu.sync_copy(typ_hbm.at[pl.ds(base, E_PER_W)], idx_all_t)

    issue(0, rs0, rd0, rr0, sem0)
    issue(1, rs1, rd1, rr1, sem1)

    def pair_body(i, _):
        c = i * 2
        drain(rs0, rd0, rr0, sem0)
        compute(rs0, rd0, rr0, c * CHUNK, CHUNK, CHUNK // 16)
        issue(c + 2, rs0, rd0, rr0, sem0)
        drain(rs1, rd1, rr1, sem1)
        compute(rs1, rd1, rr1, (c + 1) * CHUNK, CHUNK, CHUNK // 16)
        issue(c + 3, rs1, rd1, rr1, sem1)
        return 0

    lax.fori_loop(0, (N_FULL - 2) // 2, pair_body, 0)

    drain(rs0, rd0, rr0, sem0)
    compute(rs0, rd0, rr0, (N_FULL - 2) * CHUNK, CHUNK, CHUNK // 16)
    toff = N_FULL * CHUNK
    pltpu.async_copy(z_hbm.at[idx_all_s.at[pl.ds(toff, TAIL)]],
                     rs0.at[pl.ds(0, TAIL)], sem0)
    pltpu.async_copy(z_hbm.at[idx_all_d.at[pl.ds(toff, TAIL)]],
                     rd0.at[pl.ds(0, TAIL)], sem0)
    pltpu.async_copy(rel_hbm.at[idx_all_t.at[pl.ds(toff, TAIL)]],
                     rr0.at[pl.ds(0, TAIL)], sem0)

    drain(rs1, rd1, rr1, sem1)
    compute(rs1, rd1, rr1, (N_FULL - 1) * CHUNK, CHUNK, CHUNK // 16)

    pltpu.make_async_copy(z_hbm.at[idx_all_s.at[pl.ds(toff, TAIL)]],
                          rs0.at[pl.ds(0, TAIL)], sem0).wait()
    pltpu.make_async_copy(z_hbm.at[idx_all_d.at[pl.ds(toff, TAIL)]],
                          rd0.at[pl.ds(0, TAIL)], sem0).wait()
    pltpu.make_async_copy(rel_hbm.at[idx_all_t.at[pl.ds(toff, TAIL)]],
                          rr0.at[pl.ds(0, TAIL)], sem0).wait()
    compute(rs0, rd0, rr0, toff, TAIL, 1)

    pltpu.sync_copy(out_v.at[pl.ds(0, E_PER_W)],
                    out_hbm.at[pl.ds(base, E_PER_W)])


@jax.jit
def _dist_mult(src, dst, typ, z, rel_emb):
    mesh = plsc.VectorSubcoreMesh(core_axis_name="c", subcore_axis_name="s")
    rows = pltpu.VMEM((CHUNK, HIDDEN), jnp.float32)
    f = pl.kernel(
        _sc_kernel,
        out_type=jax.ShapeDtypeStruct((N_EDGES,), jnp.float32),
        mesh=mesh,
        scratch_types=[
            pltpu.VMEM((E_PER_W,), jnp.int32),
            pltpu.VMEM((E_PER_W,), jnp.int32),
            pltpu.VMEM((E_PER_W,), jnp.int32),
            rows, rows, rows, rows, rows, rows,
            pltpu.VMEM((CHUNK * 16,), jnp.float32),
            pltpu.VMEM((OUT_PAD,), jnp.float32),
            pltpu.SemaphoreType.DMA,
            pltpu.SemaphoreType.DMA,
        ],
        compiler_params=pltpu.CompilerParams(needs_layout_passes=False),
    )
    return f(src, dst, typ, z, rel_emb)


def kernel(z, edge_index, edge_type, rel_emb):
    edge_index = edge_index.astype(jnp.int32)
    edge_type = edge_type.astype(jnp.int32)
    return _dist_mult(edge_index[0], edge_index[1], edge_type, z, rel_emb)
